# baseline (speedup 1.0000x reference)
import functools

import jax
import jax.numpy as jnp
from jax import lax
from jax.experimental import pallas as pl
from jax.experimental.pallas import tpu as pltpu
from jax.experimental.pallas import tpu_sc as plsc

NC = 2
NS = 16
NW = NC * NS

_SC_PARAMS = pltpu.CompilerParams(use_tc_tiling_on_sc=False)


def _round_up(v, m):
    return ((v + m - 1) // m) * m


def _make_deg_kernel(n_pad, E, win):
    rows_per_tile = n_pad // NS
    mesh = plsc.VectorSubcoreMesh(core_axis_name="c", subcore_axis_name="s")

    @functools.partial(
        pl.kernel,
        out_type=jax.ShapeDtypeStruct((NC * n_pad, 8), jnp.float32),
        mesh=mesh,
        scratch_types=[
            pltpu.VMEM((win, 8), jnp.float32),
            pltpu.VMEM_SHARED((n_pad, 8), jnp.float32),
        ],
        compiler_params=_SC_PARAMS,
    )
    def deg_kernel(dst_h, ones_h, zeros_h, out_h, ones_v, acc):
        cid = lax.axis_index("c")
        sid = lax.axis_index("s")
        t0 = sid * rows_per_tile
        pltpu.sync_copy(zeros_h.at[pl.ds(t0, rows_per_tile)],
                        acc.at[pl.ds(t0, rows_per_tile)])
        pltpu.sync_copy(ones_h, ones_v)
        plsc.subcore_barrier()

        def body(dst_blk):
            pltpu.sync_copy(ones_v, acc.at[dst_blk.at[0]], add=True)

        pltpu.emit_pipeline(
            body,
            grid=(E // win,),
            in_specs=[pl.BlockSpec((1, win), lambda i: (0, i))],
            out_specs=[],
            core_axis_name=("c", "s"),
            dimension_semantics=(pltpu.PARALLEL,),
        )(dst_h)

        plsc.subcore_barrier()
        pltpu.sync_copy(acc.at[pl.ds(t0, rows_per_tile)],
                        out_h.at[pl.ds(cid * n_pad + t0, rows_per_tile)])

    return deg_kernel


def _make_gs_kernel(n_pad, E, win, F):
    rows_per_tile = n_pad // NS
    cells = E // win
    cells_pw = cells // NW
    mesh = plsc.VectorSubcoreMesh(core_axis_name="c", subcore_axis_name="s")

    @functools.partial(
        pl.kernel,
        out_type=jax.ShapeDtypeStruct((NC * n_pad, F), jnp.float32),
        mesh=mesh,
        scratch_types=[
            pltpu.VMEM((win, F), jnp.float32),
            pltpu.VMEM((win, F), jnp.float32),
            pltpu.VMEM((1, win), jnp.int32),
            pltpu.VMEM((1, win), jnp.int32),
            pltpu.VMEM((1, win), jnp.int32),
            pltpu.VMEM((1, win), jnp.int32),
            pltpu.VMEM_SHARED((n_pad, F), jnp.float32),
            pltpu.VMEM_SHARED((n_pad, F), jnp.float32),
            pltpu.SemaphoreType.DMA,
            pltpu.SemaphoreType.DMA,
            pltpu.SemaphoreType.DMA,
            pltpu.SemaphoreType.DMA,
        ],
        compiler_params=_SC_PARAMS,
    )
    def gs_kernel(table_h, src_h, dst_h, zeros_h, out_h,
                  rows0, rows1, s0, d0, s1, d1, acc, table_s,
                  is0, is1, g0, g1):
        cid = lax.axis_index("c")
        sid = lax.axis_index("s")
        t0 = sid * rows_per_tile
        pltpu.sync_copy(zeros_h.at[pl.ds(t0, rows_per_tile)],
                        acc.at[pl.ds(t0, rows_per_tile)])
        pltpu.sync_copy(table_h.at[pl.ds(t0, rows_per_tile)],
                        table_s.at[pl.ds(t0, rows_per_tile)])
        plsc.subcore_barrier()

        wid = cid * NS + sid
        base = wid * cells_pw
        bufs = ((s0, d0, rows0, is0, g0), (s1, d1, rows1, is1, g1))

        def stage(c, b):
            sv, dv, rv, isem, gsem = bufs[b]
            pltpu.make_async_copy(
                src_h.at[pl.ds(0, 1), pl.ds(c * win, win)], sv, isem).start()
            pltpu.make_async_copy(
                dst_h.at[pl.ds(0, 1), pl.ds(c * win, win)], dv, isem).start()
            pltpu.make_async_copy(
                src_h.at[pl.ds(0, 1), pl.ds(c * win, win)], sv, isem).wait()
            pltpu.make_async_copy(
                dst_h.at[pl.ds(0, 1), pl.ds(c * win, win)], dv, isem).wait()
            pltpu.make_async_copy(table_s.at[sv.at[0]], rv, gsem).start()

        def finish(c, b):
            sv, dv, rv, isem, gsem = bufs[b]
            pltpu.make_async_copy(table_s.at[sv.at[0]], rv, gsem).wait()
            pltpu.sync_copy(rv, acc.at[dv.at[0]], add=True)

        stage(base + 0, 0)
        stage(base + 1, 1)

        @pl.loop(0, (cells_pw - 2) // 2)
        def _(jj):
            c = base + 2 * jj
            finish(c, 0)
            stage(c + 2, 0)
            finish(c + 1, 1)
            stage(c + 3, 1)

        finish(base + cells_pw - 2, 0)
        finish(base + cells_pw - 1, 1)

        plsc.subcore_barrier()
        pltpu.sync_copy(acc.at[pl.ds(t0, rows_per_tile)],
                        out_h.at[pl.ds(cid * n_pad + t0, rows_per_tile)])

    return gs_kernel


def _tc1_body(d0_ref, d1_ref, x_ref, m4_ref, t_ref, dinv_ref):
    deg = d0_ref[...] + d1_ref[...] + 1.0
    dinv = lax.rsqrt(deg)
    dinv_ref[...] = dinv
    t_ref[...] = (x_ref[...] + m4_ref[...]) * dinv


def _tc2_body(a0_ref, a1_ref, t_ref, dinv_ref, BW1_ref, b1_ref, BW2_ref,
              pp_ref):
    dinv = dinv_ref[...]
    tt = (a0_ref[...] + a1_ref[...] + t_ref[...]) * dinv
    out1 = jnp.dot(tt, BW1_ref[...],
                   preferred_element_type=jnp.float32) + b1_ref[...]
    h2 = jnp.maximum(out1, 0.0)
    pp_ref[...] = jnp.dot(h2, BW2_ref[...],
                          preferred_element_type=jnp.float32) * dinv


def _tc3_body(a0_ref, a1_ref, pp_ref, dinv_ref, b2_ref, y_ref):
    o = (a0_ref[...] + a1_ref[...] + pp_ref[...]) * dinv_ref[...] + b2_ref[...]
    lane = lax.broadcasted_iota(jnp.int32, o.shape, 1) % 8
    partner = jnp.where(lane == 0,
                        pltpu.roll(o, 127, 1),
                        pltpu.roll(o, 1, 1))
    mx = jnp.maximum(o, partner)
    lse = mx + jnp.log(jnp.exp(o - mx) + jnp.exp(partner - mx))
    y_ref[...] = o - lse


def _blk(rows, cols, off=0):
    return pl.BlockSpec((rows, cols), lambda j, o=off: (j + o, 0))


def _full_spec(shape):
    return pl.BlockSpec(shape, lambda j: tuple(0 for _ in shape))


def kernel(x, edge_index, W1, b1, W2, b2):
    N = x.shape[0]
    E = edge_index.shape[1]
    n_pad = _round_up(N, 128)
    assert n_pad % NS == 0 and (n_pad // NS) % 8 == 0

    win = 4000
    while E % win != 0 or win % 8 != 0:
        win -= 8
    win_gs = 1000
    while E % (NW * 2 * win_gs) != 0 or win_gs % 8 != 0:
        win_gs -= 8

    k16 = n_pad // 16
    bk = 368
    while k16 % bk != 0 or bk % 8 != 0:
        bk -= 8
    grid = (k16 // bk,)
    nb = k16 // bk

    edge_index = edge_index.astype(jnp.int32)
    src = edge_index[0:1]
    dst = edge_index[1:2]
    x = x.astype(jnp.float32)
    W1 = W1.astype(jnp.float32)
    W2 = W2.astype(jnp.float32)
    b1 = b1.astype(jnp.float32)
    b2 = b2.astype(jnp.float32)

    zeros8 = jnp.zeros((n_pad // 16, 128), jnp.float32).reshape(n_pad, 8)
    ones_c = jnp.ones((win // 16, 128), jnp.float32).reshape(win, 8)

    x8i = jnp.pad(x, ((0, n_pad - N), (0, 4))).reshape(k16, 128)
    m4 = jnp.tile(jnp.array([0, 0, 0, 0, 1, 0, 0, 0], jnp.float32),
                  16).reshape(1, 128)
    BW1 = jnp.kron(jnp.eye(16, dtype=jnp.float32),
                   jnp.pad(W1, ((0, 4), (0, 0))))
    BW2 = jnp.kron(jnp.eye(16, dtype=jnp.float32),
                   jnp.pad(W2, ((0, 0), (0, 6))))
    b1t = jnp.tile(b1, 16).reshape(1, 256)
    b2t = jnp.tile(jnp.pad(b2, (0, 6)), 16).reshape(1, 128)

    deg_k = _make_deg_kernel(n_pad, E, win)
    degp = deg_k(dst, ones_c, zeros8)
    degp = degp.reshape(2 * k16, 128)

    t128, dinv8 = pl.pallas_call(
        _tc1_body,
        grid=grid,
        in_specs=[_blk(bk, 128), _blk(bk, 128, nb), _blk(bk, 128),
                  _full_spec((1, 128))],
        out_specs=[_blk(bk, 128), _blk(bk, 128)],
        out_shape=[jax.ShapeDtypeStruct((k16, 128), jnp.float32),
                   jax.ShapeDtypeStruct((k16, 128), jnp.float32)],
    )(degp, degp, x8i, m4)

    gs8 = _make_gs_kernel(n_pad, E, win_gs, 8)
    acc1 = gs8(t128.reshape(n_pad, 8), src, dst, zeros8)
    acc1 = acc1.reshape(2 * k16, 128)

    pp = pl.pallas_call(
        _tc2_body,
        grid=grid,
        in_specs=[_blk(bk, 128), _blk(bk, 128, nb), _blk(bk, 128),
                  _blk(bk, 128), _full_spec((128, 256)), _full_spec((1, 256)),
                  _full_spec((256, 128))],
        out_specs=[_blk(bk, 128)],
        out_shape=[jax.ShapeDtypeStruct((k16, 128), jnp.float32)],
    )(acc1, acc1, t128, dinv8, BW1, b1t, BW2)[0]

    acc2 = gs8(pp.reshape(n_pad, 8), src, dst, zeros8)
    acc2 = acc2.reshape(2 * k16, 128)

    y128 = pl.pallas_call(
        _tc3_body,
        grid=grid,
        in_specs=[_blk(bk, 128), _blk(bk, 128, nb), _blk(bk, 128),
                  _blk(bk, 128), _full_spec((1, 128))],
        out_specs=[_blk(bk, 128)],
        out_shape=[jax.ShapeDtypeStruct((k16, 128), jnp.float32)],
    )(acc2, acc2, pp, dinv8, b2t)[0]

    return y128.reshape(n_pad, 8)[:N, :2]


# speedup vs baseline: 1.0310x; 1.0310x over previous
import functools

import jax
import jax.numpy as jnp
from jax import lax
from jax.experimental import pallas as pl
from jax.experimental.pallas import tpu as pltpu
from jax.experimental.pallas import tpu_sc as plsc

NC = 2
NS = 16
NW = NC * NS

_SC_PARAMS = pltpu.CompilerParams(use_tc_tiling_on_sc=False)


def _round_up(v, m):
    return ((v + m - 1) // m) * m


def _make_deg_kernel(n_pad, E, win):
    rows_per_tile = n_pad // NS
    mesh = plsc.VectorSubcoreMesh(core_axis_name="c", subcore_axis_name="s")

    @functools.partial(
        pl.kernel,
        out_type=jax.ShapeDtypeStruct((NC * n_pad, 8), jnp.float32),
        mesh=mesh,
        scratch_types=[
            pltpu.VMEM((win, 8), jnp.float32),
            pltpu.VMEM_SHARED((n_pad, 8), jnp.float32),
        ],
        compiler_params=_SC_PARAMS,
    )
    def deg_kernel(dst_h, ones_h, zeros_h, out_h, ones_v, acc):
        cid = lax.axis_index("c")
        sid = lax.axis_index("s")
        t0 = sid * rows_per_tile
        pltpu.sync_copy(zeros_h.at[pl.ds(t0, rows_per_tile)],
                        acc.at[pl.ds(t0, rows_per_tile)])
        pltpu.sync_copy(ones_h, ones_v)
        plsc.subcore_barrier()

        def body(dst_blk):
            pltpu.sync_copy(ones_v, acc.at[dst_blk.at[0]], add=True)

        pltpu.emit_pipeline(
            body,
            grid=(E // win,),
            in_specs=[pl.BlockSpec((1, win), lambda i: (0, i))],
            out_specs=[],
            core_axis_name=("c", "s"),
            dimension_semantics=(pltpu.PARALLEL,),
        )(dst_h)

        plsc.subcore_barrier()
        pltpu.sync_copy(acc.at[pl.ds(t0, rows_per_tile)],
                        out_h.at[pl.ds(cid * n_pad + t0, rows_per_tile)])

    return deg_kernel


def _make_gs_kernel(n_pad, E, win, F):
    rows_per_tile = n_pad // NS
    cells = E // win
    cells_pw = cells // NW
    mesh = plsc.VectorSubcoreMesh(core_axis_name="c", subcore_axis_name="s")

    @functools.partial(
        pl.kernel,
        out_type=jax.ShapeDtypeStruct((NC * n_pad, F), jnp.float32),
        mesh=mesh,
        scratch_types=[
            pltpu.VMEM((win, F), jnp.float32),
            pltpu.VMEM((win, F), jnp.float32),
            pltpu.VMEM((1, win), jnp.int32),
            pltpu.VMEM((1, win), jnp.int32),
            pltpu.VMEM((1, win), jnp.int32),
            pltpu.VMEM((1, win), jnp.int32),
            pltpu.VMEM_SHARED((n_pad, F), jnp.float32),
            pltpu.SemaphoreType.DMA,
            pltpu.SemaphoreType.DMA,
            pltpu.SemaphoreType.DMA,
            pltpu.SemaphoreType.DMA,
        ],
        compiler_params=_SC_PARAMS,
    )
    def gs_kernel(table_h, src_h, dst_h, zeros_h, out_h,
                  rows0, rows1, s0, d0, s1, d1, acc, is0, is1, g0, g1):
        cid = lax.axis_index("c")
        sid = lax.axis_index("s")
        t0 = sid * rows_per_tile
        pltpu.sync_copy(zeros_h.at[pl.ds(t0, rows_per_tile)],
                        acc.at[pl.ds(t0, rows_per_tile)])
        plsc.subcore_barrier()

        wid = cid * NS + sid
        base = wid * cells_pw
        bufs = ((s0, d0, rows0, is0, g0), (s1, d1, rows1, is1, g1))

        def stage(c, b):
            sv, dv, rv, isem, gsem = bufs[b]
            pltpu.make_async_copy(
                src_h.at[pl.ds(0, 1), pl.ds(c * win, win)], sv, isem).start()
            pltpu.make_async_copy(
                dst_h.at[pl.ds(0, 1), pl.ds(c * win, win)], dv, isem).start()
            pltpu.make_async_copy(
                src_h.at[pl.ds(0, 1), pl.ds(c * win, win)], sv, isem).wait()
            pltpu.make_async_copy(
                dst_h.at[pl.ds(0, 1), pl.ds(c * win, win)], dv, isem).wait()
            pltpu.make_async_copy(table_h.at[sv.at[0]], rv, gsem).start()

        def finish(c, b):
            sv, dv, rv, isem, gsem = bufs[b]
            pltpu.make_async_copy(table_h.at[sv.at[0]], rv, gsem).wait()
            pltpu.sync_copy(rv, acc.at[dv.at[0]], add=True)

        stage(base + 0, 0)
        stage(base + 1, 1)

        @pl.loop(0, (cells_pw - 2) // 2)
        def _(jj):
            c = base + 2 * jj
            finish(c, 0)
            stage(c + 2, 0)
            finish(c + 1, 1)
            stage(c + 3, 1)

        finish(base + cells_pw - 2, 0)
        finish(base + cells_pw - 1, 1)

        plsc.subcore_barrier()
        pltpu.sync_copy(acc.at[pl.ds(t0, rows_per_tile)],
                        out_h.at[pl.ds(cid * n_pad + t0, rows_per_tile)])

    return gs_kernel


def _tc1_body(d0_ref, d1_ref, x_ref, m4_ref, t_ref, dinv_ref):
    deg = d0_ref[...] + d1_ref[...] + 1.0
    dinv = lax.rsqrt(deg)
    dinv_ref[...] = dinv
    t_ref[...] = (x_ref[...] + m4_ref[...]) * dinv


def _tc2_body(a0_ref, a1_ref, t_ref, dinv_ref, BW1_ref, b1_ref, BW2_ref,
              pp_ref):
    dinv = dinv_ref[...]
    tt = (a0_ref[...] + a1_ref[...] + t_ref[...]) * dinv
    out1 = jnp.dot(tt, BW1_ref[...],
                   preferred_element_type=jnp.float32) + b1_ref[...]
    h2 = jnp.maximum(out1, 0.0)
    pp_ref[...] = jnp.dot(h2, BW2_ref[...],
                          preferred_element_type=jnp.float32) * dinv


def _tc3_body(a0_ref, a1_ref, pp_ref, dinv_ref, b2_ref, y_ref):
    o = (a0_ref[...] + a1_ref[...] + pp_ref[...]) * dinv_ref[...] + b2_ref[...]
    lane = lax.broadcasted_iota(jnp.int32, o.shape, 1) % 8
    partner = jnp.where(lane == 0,
                        pltpu.roll(o, 127, 1),
                        pltpu.roll(o, 1, 1))
    mx = jnp.maximum(o, partner)
    lse = mx + jnp.log(jnp.exp(o - mx) + jnp.exp(partner - mx))
    y_ref[...] = o - lse


def _blk(rows, cols, off=0):
    return pl.BlockSpec((rows, cols), lambda j, o=off: (j + o, 0))


def _full_spec(shape):
    return pl.BlockSpec(shape, lambda j: tuple(0 for _ in shape))


def kernel(x, edge_index, W1, b1, W2, b2):
    N = x.shape[0]
    E = edge_index.shape[1]
    n_pad = _round_up(N, 128)
    assert n_pad % NS == 0 and (n_pad // NS) % 8 == 0

    win = 4000
    while E % win != 0 or win % 8 != 0:
        win -= 8
    win_gs = 2000
    while E % (NW * 2 * win_gs) != 0 or win_gs % 16 != 0:
        win_gs -= 16

    k16 = n_pad // 16
    bk = 368
    while k16 % bk != 0 or bk % 8 != 0:
        bk -= 8
    grid = (k16 // bk,)
    nb = k16 // bk

    edge_index = edge_index.astype(jnp.int32)
    src = edge_index[0:1]
    dst = edge_index[1:2]
    x = x.astype(jnp.float32)
    W1 = W1.astype(jnp.float32)
    W2 = W2.astype(jnp.float32)
    b1 = b1.astype(jnp.float32)
    b2 = b2.astype(jnp.float32)

    zeros8 = jnp.zeros((n_pad // 16, 128), jnp.float32).reshape(n_pad, 8)
    ones_c = jnp.ones((win // 16, 128), jnp.float32).reshape(win, 8)

    x8i = jnp.pad(x, ((0, n_pad - N), (0, 4))).reshape(k16, 128)
    m4 = jnp.tile(jnp.array([0, 0, 0, 0, 1, 0, 0, 0], jnp.float32),
                  16).reshape(1, 128)
    BW1 = jnp.kron(jnp.eye(16, dtype=jnp.float32),
                   jnp.pad(W1, ((0, 4), (0, 0))))
    BW2 = jnp.kron(jnp.eye(16, dtype=jnp.float32),
                   jnp.pad(W2, ((0, 0), (0, 6))))
    b1t = jnp.tile(b1, 16).reshape(1, 256)
    b2t = jnp.tile(jnp.pad(b2, (0, 6)), 16).reshape(1, 128)

    deg_k = _make_deg_kernel(n_pad, E, win)
    degp = deg_k(dst, ones_c, zeros8)
    degp = degp.reshape(2 * k16, 128)

    t128, dinv8 = pl.pallas_call(
        _tc1_body,
        grid=grid,
        in_specs=[_blk(bk, 128), _blk(bk, 128, nb), _blk(bk, 128),
                  _full_spec((1, 128))],
        out_specs=[_blk(bk, 128), _blk(bk, 128)],
        out_shape=[jax.ShapeDtypeStruct((k16, 128), jnp.float32),
                   jax.ShapeDtypeStruct((k16, 128), jnp.float32)],
    )(degp, degp, x8i, m4)

    gs8 = _make_gs_kernel(n_pad, E, win_gs, 8)
    acc1 = gs8(t128.reshape(n_pad, 8), src, dst, zeros8)
    acc1 = acc1.reshape(2 * k16, 128)

    pp = pl.pallas_call(
        _tc2_body,
        grid=grid,
        in_specs=[_blk(bk, 128), _blk(bk, 128, nb), _blk(bk, 128),
                  _blk(bk, 128), _full_spec((128, 256)), _full_spec((1, 256)),
                  _full_spec((256, 128))],
        out_specs=[_blk(bk, 128)],
        out_shape=[jax.ShapeDtypeStruct((k16, 128), jnp.float32)],
    )(acc1, acc1, t128, dinv8, BW1, b1t, BW2)[0]

    acc2 = gs8(pp.reshape(n_pad, 8), src, dst, zeros8)
    acc2 = acc2.reshape(2 * k16, 128)

    y128 = pl.pallas_call(
        _tc3_body,
        grid=grid,
        in_specs=[_blk(bk, 128), _blk(bk, 128, nb), _blk(bk, 128),
                  _blk(bk, 128), _full_spec((1, 128))],
        out_specs=[_blk(bk, 128)],
        out_shape=[jax.ShapeDtypeStruct((k16, 128), jnp.float32)],
    )(acc2, acc2, pp, dinv8, b2t)[0]

    return y128.reshape(n_pad, 8)[:N, :2]
